# Initial kernel scaffold; baseline (speedup 1.0000x reference)
import jax
import jax.numpy as jnp
from jax.experimental import pallas as pl


def kernel(news_index, user_index, news_node_dict, re_entity_adj, news_table, user_table, node_embedding, W1, b1, W2, b2, aW1, ab1, aW2, ab2, aW3, ab3, cW1, cb1, cW2, cb2, cW3, cb3):
    raise NotImplementedError("write your pallas kernel here")


# speedup vs baseline: 3.8207x; 3.8207x over previous
import functools

import jax
import jax.numpy as jnp
from jax import lax
from jax.experimental import pallas as pl
from jax.experimental.pallas import tpu as pltpu
from jax.experimental.pallas import tpu_sc as plsc

_B = 4096
_NPN = 12
_ADJ = 20
_D = 100
_DP = 128
_TITLE = 400
_TITLEP = 512
_NW = 32
_BPW = _B // _NW
_SLOTS = _BPW * _NPN
_CS = 16


def _sc_gather_body(nidx, uidx, ndict, adjf, embp, ntab, utab,
                    news_g, user_g, preact,
                    ni, ui, npos, nodes, apos, nvals,
                    gbuf, nbuf, obuf, newsb, userb, sem, sem2):
  wid = lax.axis_index("s") * 2 + lax.axis_index("c")
  base = wid * _BPW

  pltpu.sync_copy(nidx.at[pl.ds(base, _BPW)], ni)
  pltpu.sync_copy(uidx.at[pl.ds(base, _BPW)], ui)

  def astep(i, _):
    niv = ni[pl.ds(i * 16, 16)]
    for j in range(_NPN):
      npos[pl.ds(j * _BPW + i * 16, 16)] = niv * _NPN + j
    return 0
  lax.fori_loop(0, _BPW // 16, astep, 0, unroll=False)

  pltpu.async_copy(ndict.at[npos], nodes, sem).wait()

  def cstep(i, _):
    nv = nodes[pl.ds(i * 16, 16)]
    for j in range(_ADJ):
      apos[pl.ds(i * (16 * _ADJ) + j * 16, 16)] = nv * _ADJ + j
    return 0
  lax.fori_loop(0, _SLOTS // 16, cstep, 0, unroll=False)
  pltpu.async_copy(adjf.at[apos], nvals, sem).wait()

  def estep(i, _):
    pltpu.async_copy(ntab.at[ni.at[pl.ds(i * 8, 8)]], newsb, sem).wait()
    pltpu.sync_copy(newsb, news_g.at[pl.ds(base + i * 8, 8)])
    pltpu.async_copy(utab.at[ui.at[pl.ds(i * 8, 8)]], userb, sem).wait()
    pltpu.sync_copy(userb, user_g.at[pl.ds(base + i * 8, 8)])
    return 0
  lax.fori_loop(0, _BPW // 8, estep, 0, unroll=False)

  def dstep(c, _):
    off = c * _CS
    cp1 = pltpu.async_copy(embp.at[nvals.at[pl.ds(off * _ADJ, _CS * _ADJ)]],
                           gbuf, sem)
    cp2 = pltpu.async_copy(embp.at[nodes.at[pl.ds(off, _CS)]], nbuf, sem2)
    cp1.wait()
    cp2.wait()
    def sstep(s, _):
      for v in range(_DP // 16):
        sl = pl.ds(v * 16, 16)
        acc = gbuf[s, sl]
        for j in range(1, _ADJ):
          acc = acc + gbuf[j * _CS + s, sl]
        obuf[s, sl] = nbuf[s, sl] + acc * (1.0 / _ADJ)
      return 0
    lax.fori_loop(0, _CS, sstep, 0, unroll=False)
    jrow = c // (_BPW // _CS)
    b0 = (c % (_BPW // _CS)) * _CS
    pltpu.sync_copy(obuf, preact.at[pl.ds(jrow * _B + base + b0, _CS)])
    return 0
  lax.fori_loop(0, _SLOTS // _CS, dstep, 0, unroll=False)


def _sc_gather(news_index, user_index, ndict_flat, adj_flat, embp,
               news_table, user_table):
  mesh = plsc.VectorSubcoreMesh(core_axis_name="c", subcore_axis_name="s",
                                num_cores=2, num_subcores=16)
  f = pl.kernel(
      _sc_gather_body,
      out_type=(
          jax.ShapeDtypeStruct((_B, _TITLEP), jnp.float32),
          jax.ShapeDtypeStruct((_B, _DP), jnp.float32),
          jax.ShapeDtypeStruct((_B * _NPN, _DP), jnp.float32),
      ),
      mesh=mesh,
      scratch_types=(
          pltpu.VMEM((_BPW,), jnp.int32),
          pltpu.VMEM((_BPW,), jnp.int32),
          pltpu.VMEM((_SLOTS,), jnp.int32),
          pltpu.VMEM((_SLOTS,), jnp.int32),
          pltpu.VMEM((_SLOTS * _ADJ,), jnp.int32),
          pltpu.VMEM((_SLOTS * _ADJ,), jnp.int32),
          pltpu.VMEM((_CS * _ADJ, _DP), jnp.float32),
          pltpu.VMEM((_CS, _DP), jnp.float32),
          pltpu.VMEM((_CS, _DP), jnp.float32),
          pltpu.VMEM((8, _TITLEP), jnp.float32),
          pltpu.VMEM((8, _DP), jnp.float32),
          pltpu.SemaphoreType.DMA,
          pltpu.SemaphoreType.DMA,
      ),
  )
  return f(news_index, user_index, ndict_flat, adj_flat, embp,
           news_table, user_table)


_BM = 256


def _tc_mlp_body(news_ref, user_ref, pre_ref,
                 W1, b1, W2, b2, aW1t, aW1b, ab1, aW2, ab2, a3, ab3,
                 cW1t, cW1b, cb1, cW2, cb2, c3, cb3,
                 out_a, out_c):
  t = jnp.dot(news_ref[...], W1[...],
              preferred_element_type=jnp.float32) + b1[...]
  t = jnp.where(t > 0, t, jnp.exp(t) - 1.0)
  ne = jnp.tanh(jnp.dot(t, W2[...],
                        preferred_element_type=jnp.float32) + b2[...])
  state = jnp.concatenate([ne, user_ref[:, :_D]], axis=1)
  sA = jnp.dot(state, aW1t[...],
               preferred_element_type=jnp.float32) + ab1[...]
  sC = jnp.dot(state, cW1t[...],
               preferred_element_type=jnp.float32) + cb1[...]
  sA12 = jnp.broadcast_to(sA[None, :, :], (_NPN, _BM, _D)).reshape(
      _NPN * _BM, _D)
  sC12 = jnp.broadcast_to(sC[None, :, :], (_NPN, _BM, _D)).reshape(
      _NPN * _BM, _D)

  act = jnp.tanh(pre_ref[...].reshape(_NPN * _BM, _DP)[:, :_D])

  ha = jnp.tanh(jnp.dot(act, aW1b[...],
                        preferred_element_type=jnp.float32) + sA12)
  oa = jnp.tanh(jnp.dot(ha, aW2[...],
                        preferred_element_type=jnp.float32) + ab2[...])
  pa = jax.nn.sigmoid(jnp.sum(oa * a3[...], axis=1) + ab3[0, 0])
  out_a[...] = pa.reshape(_NPN, _BM)

  hc = jnp.tanh(jnp.dot(act, cW1b[...],
                        preferred_element_type=jnp.float32) + sC12)
  oc = jnp.tanh(jnp.dot(hc, cW2[...],
                        preferred_element_type=jnp.float32) + cb2[...])
  pc = jax.nn.sigmoid(jnp.sum(oc * c3[...], axis=1) + cb3[0, 0])
  out_c[...] = pc.reshape(_NPN, _BM)


def _tc_mlp(news_g, user_g, pre3, W1, b1, W2, b2,
            aW1t, aW1b, ab1, aW2, ab2, a3, ab3,
            cW1t, cW1b, cb1, cW2, cb2, c3, cb3):
  nb = _B // _BM
  full = lambda shape: pl.BlockSpec(shape, lambda b: (0,) * len(shape))
  return pl.pallas_call(
      _tc_mlp_body,
      grid=(nb,),
      in_specs=[
          pl.BlockSpec((_BM, _TITLEP), lambda b: (b, 0)),
          pl.BlockSpec((_BM, _DP), lambda b: (b, 0)),
          pl.BlockSpec((_NPN, _BM, _DP), lambda b: (0, b, 0)),
          full((_TITLEP, _D)), full((1, _D)), full((_D, _D)), full((1, _D)),
          full((2 * _D, _D)), full((_D, _D)), full((1, _D)),
          full((_D, _D)), full((1, _D)), full((1, _D)), full((1, 1)),
          full((2 * _D, _D)), full((_D, _D)), full((1, _D)),
          full((_D, _D)), full((1, _D)), full((1, _D)), full((1, 1)),
      ],
      out_specs=[
          pl.BlockSpec((_NPN, _BM), lambda b: (0, b)),
          pl.BlockSpec((_NPN, _BM), lambda b: (0, b)),
      ],
      out_shape=(
          jax.ShapeDtypeStruct((_NPN, _B), jnp.float32),
          jax.ShapeDtypeStruct((_NPN, _B), jnp.float32),
      ),
  )(news_g, user_g, pre3, W1, b1, W2, b2,
    aW1t, aW1b, ab1, aW2, ab2, a3, ab3,
    cW1t, cW1b, cb1, cW2, cb2, c3, cb3)


def kernel(news_index, user_index, news_node_dict, re_entity_adj, news_table,
           user_table, node_embedding, W1, b1, W2, b2, aW1, ab1, aW2, ab2,
           aW3, ab3, cW1, cb1, cW2, cb2, cW3, cb3):
  ni = news_index.astype(jnp.int32)
  ui = user_index.astype(jnp.int32)
  ndict_flat = news_node_dict.astype(jnp.int32).reshape(-1)
  adj_flat = re_entity_adj.astype(jnp.int32).reshape(-1)
  embp = jnp.pad(node_embedding, ((0, 0), (0, _DP - _D)))
  ntabp = jnp.pad(news_table, ((0, 0), (0, _TITLEP - _TITLE)))
  utabp = jnp.pad(user_table, ((0, 0), (0, _DP - _D)))
  W1p = jnp.pad(W1, ((0, _TITLEP - _TITLE), (0, 0)))

  news_g, user_g, preact = _sc_gather(ni, ui, ndict_flat, adj_flat, embp,
                                      ntabp, utabp)
  pre3 = preact.reshape(_NPN, _B, _DP)

  out_a, out_c = _tc_mlp(
      news_g, user_g, pre3,
      W1p, b1.reshape(1, _D), W2, b2.reshape(1, _D),
      aW1[: 2 * _D], aW1[2 * _D:], ab1.reshape(1, _D),
      aW2, ab2.reshape(1, _D), aW3.reshape(1, _D), ab3.reshape(1, 1),
      cW1[: 2 * _D], cW1[2 * _D:], cb1.reshape(1, _D),
      cW2, cb2.reshape(1, _D), cW3.reshape(1, _D), cb3.reshape(1, 1),
  )
  act_probs = out_a.T.reshape(_B, _NPN, 1)
  q_actions = out_c.T.reshape(_B, _NPN, 1)
  return (act_probs, q_actions)


# speedup vs baseline: 5.4726x; 1.4324x over previous
import functools

import jax
import jax.numpy as jnp
from jax import lax
from jax.experimental import pallas as pl
from jax.experimental.pallas import tpu as pltpu
from jax.experimental.pallas import tpu_sc as plsc

_B = 4096
_NPN = 12
_ADJ = 20
_D = 100
_DP = 128
_TITLE = 400
_TITLEP = 512
_NW = 32
_BPW = _B // _NW
_SLOTS = _BPW * _NPN
_CS = 16


def _sc_gather_body(nidx, uidx, ndict, adjf, embp, ntab, utab,
                    news_g, user_g, preact,
                    ni, ui, npos, nodes, apos, nvals,
                    gbuf, nbuf, obuf, newsb, userb, sem, sem2):
  wid = lax.axis_index("s") * 2 + lax.axis_index("c")
  base = wid * _BPW

  pltpu.sync_copy(nidx.at[pl.ds(base, _BPW)], ni)
  pltpu.sync_copy(uidx.at[pl.ds(base, _BPW)], ui)

  def astep(i, _):
    niv = ni[pl.ds(i * 16, 16)]
    for j in range(_NPN):
      npos[pl.ds(j * _BPW + i * 16, 16)] = niv * _NPN + j
    return 0
  lax.fori_loop(0, _BPW // 16, astep, 0, unroll=False)

  pltpu.async_copy(ndict.at[npos], nodes, sem).wait()

  def cstep(i, _):
    nv = nodes[pl.ds(i * 16, 16)]
    for j in range(_ADJ):
      apos[pl.ds(i * (16 * _ADJ) + j * 16, 16)] = nv * _ADJ + j
    return 0
  lax.fori_loop(0, _SLOTS // 16, cstep, 0, unroll=False)
  pltpu.async_copy(adjf.at[apos], nvals, sem).wait()

  def estep(i, _):
    pltpu.async_copy(ntab.at[ni.at[pl.ds(i * 8, 8)]], newsb, sem).wait()
    pltpu.sync_copy(newsb, news_g.at[pl.ds(base + i * 8, 8)])
    pltpu.async_copy(utab.at[ui.at[pl.ds(i * 8, 8)]], userb, sem).wait()
    pltpu.sync_copy(userb, user_g.at[pl.ds(base + i * 8, 8)])
    return 0
  lax.fori_loop(0, _BPW // 8, estep, 0, unroll=False)

  def dstep(c, _):
    off = c * _CS
    cp1 = pltpu.async_copy(embp.at[nvals.at[pl.ds(off * _ADJ, _CS * _ADJ)]],
                           gbuf, sem)
    cp2 = pltpu.async_copy(embp.at[nodes.at[pl.ds(off, _CS)]], nbuf, sem2)
    cp1.wait()
    cp2.wait()
    def sstep(s, _):
      for v in range(_D // 16 + 1):
        sl = pl.ds(v * 16, 16)
        acc = gbuf[s, sl]
        for j in range(1, _ADJ):
          acc = acc + gbuf[j * _CS + s, sl]
        obuf[s, sl] = nbuf[s, sl] + acc * (1.0 / _ADJ)
      return 0
    lax.fori_loop(0, _CS, sstep, 0, unroll=False)
    jrow = c // (_BPW // _CS)
    b0 = (c % (_BPW // _CS)) * _CS
    pltpu.sync_copy(obuf, preact.at[pl.ds(jrow * _B + base + b0, _CS)])
    return 0
  lax.fori_loop(0, _SLOTS // _CS, dstep, 0, unroll=False)


def _sc_gather(news_index, user_index, ndict_flat, adj_flat, embp,
               news_table, user_table):
  mesh = plsc.VectorSubcoreMesh(core_axis_name="c", subcore_axis_name="s",
                                num_cores=2, num_subcores=16)
  f = pl.kernel(
      _sc_gather_body,
      out_type=(
          jax.ShapeDtypeStruct((_B, _TITLEP), jnp.float32),
          jax.ShapeDtypeStruct((_B, _DP), jnp.float32),
          jax.ShapeDtypeStruct((_B * _NPN, _DP), jnp.float32),
      ),
      mesh=mesh,
      scratch_types=(
          pltpu.VMEM((_BPW,), jnp.int32),
          pltpu.VMEM((_BPW,), jnp.int32),
          pltpu.VMEM((_SLOTS,), jnp.int32),
          pltpu.VMEM((_SLOTS,), jnp.int32),
          pltpu.VMEM((_SLOTS * _ADJ,), jnp.int32),
          pltpu.VMEM((_SLOTS * _ADJ,), jnp.int32),
          pltpu.VMEM((_CS * _ADJ, _DP), jnp.float32),
          pltpu.VMEM((_CS, _DP), jnp.float32),
          pltpu.VMEM((_CS, _DP), jnp.float32),
          pltpu.VMEM((8, _TITLEP), jnp.float32),
          pltpu.VMEM((8, _DP), jnp.float32),
          pltpu.SemaphoreType.DMA,
          pltpu.SemaphoreType.DMA,
      ),
  )
  return f(news_index, user_index, ndict_flat, adj_flat, embp,
           news_table, user_table)


def _tc_pad(x, out_cols, rows_per_block):
  rows, in_cols = x.shape
  nblk = (rows + rows_per_block - 1) // rows_per_block

  def body(x_ref, o_ref):
    o_ref[...] = jnp.concatenate(
        [x_ref[...],
         jnp.zeros((rows_per_block, out_cols - in_cols), jnp.float32)],
        axis=1)

  return pl.pallas_call(
      body,
      grid=(nblk,),
      in_specs=[pl.BlockSpec((rows_per_block, in_cols), lambda b: (b, 0))],
      out_specs=pl.BlockSpec((rows_per_block, out_cols), lambda b: (b, 0)),
      out_shape=jax.ShapeDtypeStruct((rows, out_cols), jnp.float32),
  )(x)


_BM = 256


def _tc_mlp_body(news_ref, user_ref, pre_ref,
                 W1, b1, W2, b2, aW1t, aW1b, ab1, aW2, ab2, a3, ab3,
                 cW1t, cW1b, cb1, cW2, cb2, c3, cb3,
                 out_a, out_c):
  t = jnp.dot(news_ref[...], W1[...],
              preferred_element_type=jnp.float32) + b1[...]
  t = jnp.where(t > 0, t, jnp.exp(t) - 1.0)
  ne = jnp.tanh(jnp.dot(t, W2[...],
                        preferred_element_type=jnp.float32) + b2[...])
  state = jnp.concatenate([ne, user_ref[:, :_D]], axis=1)
  sA = jnp.dot(state, aW1t[...],
               preferred_element_type=jnp.float32) + ab1[...]
  sC = jnp.dot(state, cW1t[...],
               preferred_element_type=jnp.float32) + cb1[...]
  sA12 = jnp.broadcast_to(sA[None, :, :], (_NPN, _BM, _D)).reshape(
      _NPN * _BM, _D)
  sC12 = jnp.broadcast_to(sC[None, :, :], (_NPN, _BM, _D)).reshape(
      _NPN * _BM, _D)

  act = jnp.tanh(pre_ref[...].reshape(_NPN * _BM, _DP)[:, :_D])

  ha = jnp.tanh(jnp.dot(act, aW1b[...],
                        preferred_element_type=jnp.float32) + sA12)
  oa = jnp.tanh(jnp.dot(ha, aW2[...],
                        preferred_element_type=jnp.float32) + ab2[...])
  pa = jax.nn.sigmoid(jnp.sum(oa * a3[...], axis=1) + ab3[0, 0])
  out_a[...] = pa.reshape(_NPN, _BM)

  hc = jnp.tanh(jnp.dot(act, cW1b[...],
                        preferred_element_type=jnp.float32) + sC12)
  oc = jnp.tanh(jnp.dot(hc, cW2[...],
                        preferred_element_type=jnp.float32) + cb2[...])
  pc = jax.nn.sigmoid(jnp.sum(oc * c3[...], axis=1) + cb3[0, 0])
  out_c[...] = pc.reshape(_NPN, _BM)


def _tc_mlp(news_g, user_g, pre3, W1, b1, W2, b2,
            aW1t, aW1b, ab1, aW2, ab2, a3, ab3,
            cW1t, cW1b, cb1, cW2, cb2, c3, cb3):
  nb = _B // _BM
  full = lambda shape: pl.BlockSpec(shape, lambda b: (0,) * len(shape))
  return pl.pallas_call(
      _tc_mlp_body,
      grid=(nb,),
      in_specs=[
          pl.BlockSpec((_BM, _TITLEP), lambda b: (b, 0)),
          pl.BlockSpec((_BM, _DP), lambda b: (b, 0)),
          pl.BlockSpec((_NPN, _BM, _DP), lambda b: (0, b, 0)),
          full((_TITLEP, _D)), full((1, _D)), full((_D, _D)), full((1, _D)),
          full((2 * _D, _D)), full((_D, _D)), full((1, _D)),
          full((_D, _D)), full((1, _D)), full((1, _D)), full((1, 1)),
          full((2 * _D, _D)), full((_D, _D)), full((1, _D)),
          full((_D, _D)), full((1, _D)), full((1, _D)), full((1, 1)),
      ],
      out_specs=[
          pl.BlockSpec((_NPN, _BM), lambda b: (0, b)),
          pl.BlockSpec((_NPN, _BM), lambda b: (0, b)),
      ],
      out_shape=(
          jax.ShapeDtypeStruct((_NPN, _B), jnp.float32),
          jax.ShapeDtypeStruct((_NPN, _B), jnp.float32),
      ),
  )(news_g, user_g, pre3, W1, b1, W2, b2,
    aW1t, aW1b, ab1, aW2, ab2, a3, ab3,
    cW1t, cW1b, cb1, cW2, cb2, c3, cb3)


def kernel(news_index, user_index, news_node_dict, re_entity_adj, news_table,
           user_table, node_embedding, W1, b1, W2, b2, aW1, ab1, aW2, ab2,
           aW3, ab3, cW1, cb1, cW2, cb2, cW3, cb3):
  ni = news_index.astype(jnp.int32)
  ui = user_index.astype(jnp.int32)
  ndict_flat = jnp.maximum(news_node_dict.astype(jnp.int32).reshape(-1), 0)
  adj_flat = jnp.maximum(re_entity_adj.astype(jnp.int32).reshape(-1), 0)
  embp = _tc_pad(node_embedding, _DP, 2048)
  ntabp = _tc_pad(news_table, _TITLEP, 2000)
  utabp = _tc_pad(user_table, _DP, 2000)
  W1p = jnp.pad(W1, ((0, _TITLEP - _TITLE), (0, 0)))

  news_g, user_g, preact = _sc_gather(ni, ui, ndict_flat, adj_flat, embp,
                                      ntabp, utabp)
  pre3 = preact.reshape(_NPN, _B, _DP)

  out_a, out_c = _tc_mlp(
      news_g, user_g, pre3,
      W1p, b1.reshape(1, _D), W2, b2.reshape(1, _D),
      aW1[: 2 * _D], aW1[2 * _D:], ab1.reshape(1, _D),
      aW2, ab2.reshape(1, _D), aW3.reshape(1, _D), ab3.reshape(1, 1),
      cW1[: 2 * _D], cW1[2 * _D:], cb1.reshape(1, _D),
      cW2, cb2.reshape(1, _D), cW3.reshape(1, _D), cb3.reshape(1, 1),
  )
  act_probs = out_a.T.reshape(_B, _NPN, 1)
  q_actions = out_c.T.reshape(_B, _NPN, 1)
  return (act_probs, q_actions)


# speedup vs baseline: 6.0934x; 1.1134x over previous
import functools

import jax
import jax.numpy as jnp
from jax import lax
from jax.experimental import pallas as pl
from jax.experimental.pallas import tpu as pltpu
from jax.experimental.pallas import tpu_sc as plsc

_B = 4096
_NPN = 12
_ADJ = 20
_D = 100
_DP = 128
_TITLE = 400
_TITLEP = 512
_NW = 32
_BPW = _B // _NW
_SLOTS = _BPW * _NPN
_CS = 16


def _sc_gather_body(nidx, uidx, ndict, adjf, embp, ntab, utab,
                    news_g, user_g, preact,
                    ni, ui, npos, nodes, apos, nvals,
                    gbuf, nbuf, obuf, newsb, userb,
                    sem, sem_e, sems_adj, sems_g, sems_n, sems_o):
  wid = lax.axis_index("s") * 2 + lax.axis_index("c")
  base = wid * _BPW

  pltpu.sync_copy(nidx.at[pl.ds(base, _BPW)], ni)
  pltpu.sync_copy(uidx.at[pl.ds(base, _BPW)], ui)

  def astep(i, _):
    niv = ni[pl.ds(i * 16, 16)]
    for j in range(_NPN):
      npos[pl.ds(j * _BPW + i * 16, 16)] = niv * _NPN + j
    return 0
  lax.fori_loop(0, _BPW // 16, astep, 0, unroll=False)

  pltpu.async_copy(ndict.at[npos], nodes, sem).wait()

  def estep(i, _):
    pltpu.async_copy(ntab.at[ni.at[pl.ds(i * 8, 8)]], newsb, sem_e).wait()
    pltpu.sync_copy(newsb, news_g.at[pl.ds(base + i * 8, 8)])
    pltpu.async_copy(utab.at[ui.at[pl.ds(i * 8, 8)]], userb, sem_e).wait()
    pltpu.sync_copy(userb, user_g.at[pl.ds(base + i * 8, 8)])
    return 0
  lax.fori_loop(0, _BPW // 8, estep, 0, unroll=False)

  NCH = _SLOTS // _CS

  def build_and_fetch_adj(c):
    k3 = lax.rem(c, 3)
    nv = nodes[pl.ds(c * _CS, _CS)]
    for j in range(_ADJ):
      apos[pl.ds(k3 * (_CS * _ADJ) + j * _CS, _CS)] = nv * _ADJ + j
    pltpu.async_copy(adjf.at[apos.at[pl.ds(k3 * (_CS * _ADJ), _CS * _ADJ)]],
                     nvals.at[pl.ds(k3 * (_CS * _ADJ), _CS * _ADJ)],
                     sems_adj.at[k3])

  def fetch_emb(c):
    k3 = lax.rem(c, 3)
    k2 = lax.rem(c, 2)
    pltpu.make_async_copy(
        adjf.at[apos.at[pl.ds(k3 * (_CS * _ADJ), _CS * _ADJ)]],
        nvals.at[pl.ds(k3 * (_CS * _ADJ), _CS * _ADJ)],
        sems_adj.at[k3]).wait()
    pltpu.async_copy(embp.at[nvals.at[pl.ds(k3 * (_CS * _ADJ), _CS * _ADJ)]],
                     gbuf.at[k2], sems_g.at[k2])
    pltpu.async_copy(embp.at[nodes.at[pl.ds(c * _CS, _CS)]],
                     nbuf.at[k2], sems_n.at[k2])

  def out_copy(c):
    k2 = lax.rem(c, 2)
    jrow = c // (_BPW // _CS)
    b0 = lax.rem(c, _BPW // _CS) * _CS
    return pltpu.make_async_copy(
        obuf.at[k2], preact.at[jrow, pl.ds(base + b0, _CS)], sems_o.at[k2])

  build_and_fetch_adj(0)
  build_and_fetch_adj(1)
  fetch_emb(0)

  def dstep(c, _):
    k2 = lax.rem(c, 2)
    k3 = lax.rem(c, 3)

    @pl.when(c + 2 < NCH)
    def _():
      build_and_fetch_adj(c + 2)

    @pl.when(c + 1 < NCH)
    def _():
      fetch_emb(c + 1)

    pltpu.make_async_copy(
        embp.at[nvals.at[pl.ds(k3 * (_CS * _ADJ), _CS * _ADJ)]],
        gbuf.at[k2], sems_g.at[k2]).wait()
    pltpu.make_async_copy(embp.at[nodes.at[pl.ds(c * _CS, _CS)]],
                          nbuf.at[k2], sems_n.at[k2]).wait()

    @pl.when(c >= 2)
    def _():
      out_copy(c - 2).wait()

    def sstep(s, _):
      for v in range(_D // 16 + 1):
        sl = pl.ds(v * 16, 16)
        acc = gbuf[k2, s, sl]
        for j in range(1, _ADJ):
          acc = acc + gbuf[k2, j * _CS + s, sl]
        obuf[k2, s, sl] = nbuf[k2, s, sl] + acc * (1.0 / _ADJ)
      return 0
    lax.fori_loop(0, _CS, sstep, 0, unroll=False)
    out_copy(c).start()
    return 0
  lax.fori_loop(0, NCH, dstep, 0, unroll=False)
  out_copy(NCH - 2).wait()
  out_copy(NCH - 1).wait()


def _sc_gather(news_index, user_index, ndict_flat, adj_flat, embp,
               news_table, user_table):
  mesh = plsc.VectorSubcoreMesh(core_axis_name="c", subcore_axis_name="s",
                                num_cores=2, num_subcores=16)
  f = pl.kernel(
      _sc_gather_body,
      out_type=(
          jax.ShapeDtypeStruct((_B, _TITLEP), jnp.float32),
          jax.ShapeDtypeStruct((_B, _DP), jnp.float32),
          jax.ShapeDtypeStruct((_NPN, _B, _DP), jnp.float32),
      ),
      mesh=mesh,
      scratch_types=(
          pltpu.VMEM((_BPW,), jnp.int32),
          pltpu.VMEM((_BPW,), jnp.int32),
          pltpu.VMEM((_SLOTS,), jnp.int32),
          pltpu.VMEM((_SLOTS,), jnp.int32),
          pltpu.VMEM((3 * _CS * _ADJ,), jnp.int32),
          pltpu.VMEM((3 * _CS * _ADJ,), jnp.int32),
          pltpu.VMEM((2, _CS * _ADJ, _DP), jnp.float32),
          pltpu.VMEM((2, _CS, _DP), jnp.float32),
          pltpu.VMEM((2, _CS, _DP), jnp.float32),
          pltpu.VMEM((8, _TITLEP), jnp.float32),
          pltpu.VMEM((8, _DP), jnp.float32),
          pltpu.SemaphoreType.DMA,
          pltpu.SemaphoreType.DMA,
          pltpu.SemaphoreType.DMA((3,)),
          pltpu.SemaphoreType.DMA((2,)),
          pltpu.SemaphoreType.DMA((2,)),
          pltpu.SemaphoreType.DMA((2,)),
      ),
  )
  return f(news_index, user_index, ndict_flat, adj_flat, embp,
           news_table, user_table)


def _tc_pad(x, out_cols, rows_per_block):
  rows, in_cols = x.shape
  nblk = (rows + rows_per_block - 1) // rows_per_block

  def body(x_ref, o_ref):
    o_ref[...] = jnp.concatenate(
        [x_ref[...],
         jnp.zeros((rows_per_block, out_cols - in_cols), jnp.float32)],
        axis=1)

  return pl.pallas_call(
      body,
      grid=(nblk,),
      in_specs=[pl.BlockSpec((rows_per_block, in_cols), lambda b: (b, 0))],
      out_specs=pl.BlockSpec((rows_per_block, out_cols), lambda b: (b, 0)),
      out_shape=jax.ShapeDtypeStruct((rows, out_cols), jnp.float32),
  )(x)


_BM = 256


def _tc_mlp_body(news_ref, user_ref, pre_ref,
                 W1, b1, W2, b2, aW1t, aW1b, ab1, aW2, ab2, a3, ab3,
                 cW1t, cW1b, cb1, cW2, cb2, c3, cb3,
                 out_a, out_c):
  t = jnp.dot(news_ref[:, :_TITLE], W1[...],
              preferred_element_type=jnp.float32) + b1[...]
  t = jnp.where(t > 0, t, jnp.exp(t) - 1.0)
  ne = jnp.tanh(jnp.dot(t, W2[...],
                        preferred_element_type=jnp.float32) + b2[...])
  state = jnp.concatenate([ne, user_ref[:, :_D]], axis=1)
  sA = jnp.dot(state, aW1t[...],
               preferred_element_type=jnp.float32) + ab1[...]
  sC = jnp.dot(state, cW1t[...],
               preferred_element_type=jnp.float32) + cb1[...]
  sA12 = jnp.broadcast_to(sA[None, :, :], (_NPN, _BM, _D)).reshape(
      _NPN * _BM, _D)
  sC12 = jnp.broadcast_to(sC[None, :, :], (_NPN, _BM, _D)).reshape(
      _NPN * _BM, _D)

  act = jnp.tanh(pre_ref[...].reshape(_NPN * _BM, _DP)[:, :_D])

  ha = jnp.tanh(jnp.dot(act, aW1b[...],
                        preferred_element_type=jnp.float32) + sA12)
  oa = jnp.tanh(jnp.dot(ha, aW2[...],
                        preferred_element_type=jnp.float32) + ab2[...])
  pa = jax.nn.sigmoid(jnp.sum(oa * a3[...], axis=1) + ab3[0, 0])
  out_a[...] = pa.reshape(_NPN, _BM)

  hc = jnp.tanh(jnp.dot(act, cW1b[...],
                        preferred_element_type=jnp.float32) + sC12)
  oc = jnp.tanh(jnp.dot(hc, cW2[...],
                        preferred_element_type=jnp.float32) + cb2[...])
  pc = jax.nn.sigmoid(jnp.sum(oc * c3[...], axis=1) + cb3[0, 0])
  out_c[...] = pc.reshape(_NPN, _BM)


def _tc_mlp(news_g, user_g, pre3, W1, b1, W2, b2,
            aW1t, aW1b, ab1, aW2, ab2, a3, ab3,
            cW1t, cW1b, cb1, cW2, cb2, c3, cb3):
  nb = _B // _BM
  full = lambda shape: pl.BlockSpec(shape, lambda b: (0,) * len(shape))
  return pl.pallas_call(
      _tc_mlp_body,
      grid=(nb,),
      in_specs=[
          pl.BlockSpec((_BM, _TITLEP), lambda b: (b, 0)),
          pl.BlockSpec((_BM, _DP), lambda b: (b, 0)),
          pl.BlockSpec((_NPN, _BM, _DP), lambda b: (0, b, 0)),
          full((_TITLE, _D)), full((1, _D)), full((_D, _D)), full((1, _D)),
          full((2 * _D, _D)), full((_D, _D)), full((1, _D)),
          full((_D, _D)), full((1, _D)), full((1, _D)), full((1, 1)),
          full((2 * _D, _D)), full((_D, _D)), full((1, _D)),
          full((_D, _D)), full((1, _D)), full((1, _D)), full((1, 1)),
      ],
      out_specs=[
          pl.BlockSpec((_NPN, _BM), lambda b: (0, b)),
          pl.BlockSpec((_NPN, _BM), lambda b: (0, b)),
      ],
      out_shape=(
          jax.ShapeDtypeStruct((_NPN, _B), jnp.float32),
          jax.ShapeDtypeStruct((_NPN, _B), jnp.float32),
      ),
  )(news_g, user_g, pre3, W1, b1, W2, b2,
    aW1t, aW1b, ab1, aW2, ab2, a3, ab3,
    cW1t, cW1b, cb1, cW2, cb2, c3, cb3)


def kernel(news_index, user_index, news_node_dict, re_entity_adj, news_table,
           user_table, node_embedding, W1, b1, W2, b2, aW1, ab1, aW2, ab2,
           aW3, ab3, cW1, cb1, cW2, cb2, cW3, cb3):
  ni = news_index.astype(jnp.int32)
  ui = user_index.astype(jnp.int32)
  ndict_flat = jnp.maximum(news_node_dict.astype(jnp.int32).reshape(-1), 0)
  adj_flat = jnp.maximum(re_entity_adj.astype(jnp.int32).reshape(-1), 0)
  embp = _tc_pad(node_embedding, _DP, 2048)
  ntabp = _tc_pad(news_table, _TITLEP, 2000)
  utabp = _tc_pad(user_table, _DP, 2000)

  news_g, user_g, pre3 = _sc_gather(ni, ui, ndict_flat, adj_flat, embp,
                                    ntabp, utabp)

  out_a, out_c = _tc_mlp(
      news_g, user_g, pre3,
      W1, b1.reshape(1, _D), W2, b2.reshape(1, _D),
      aW1[: 2 * _D], aW1[2 * _D:], ab1.reshape(1, _D),
      aW2, ab2.reshape(1, _D), aW3.reshape(1, _D), ab3.reshape(1, 1),
      cW1[: 2 * _D], cW1[2 * _D:], cb1.reshape(1, _D),
      cW2, cb2.reshape(1, _D), cW3.reshape(1, _D), cb3.reshape(1, 1),
  )
  act_probs = out_a.T.reshape(_B, _NPN, 1)
  q_actions = out_c.T.reshape(_B, _NPN, 1)
  return (act_probs, q_actions)


# speedup vs baseline: 7.5269x; 1.2353x over previous
import functools

import jax
import jax.numpy as jnp
from jax import lax
from jax.experimental import pallas as pl
from jax.experimental.pallas import tpu as pltpu
from jax.experimental.pallas import tpu_sc as plsc

_B = 4096
_NPN = 12
_ADJ = 20
_D = 100
_DP = 128
_TITLE = 400
_TITLEP = 512
_NW = 32
_BPW = _B // _NW
_SLOTS = _BPW * _NPN
_CS = 16


def _sc_gather_body(nidx, uidx, ndict, adjf, embp, ntab, utab,
                    news_g, user_g, preact,
                    ni, ui, npos, nodes, apos, nvals,
                    gbuf, nbuf, obuf, newsb, userb,
                    sem, sem_e, sems_adj, sems_g, sems_n, sems_o):
  wid = lax.axis_index("s") * 2 + lax.axis_index("c")
  base = wid * _BPW

  pltpu.sync_copy(nidx.at[pl.ds(base, _BPW)], ni)
  pltpu.sync_copy(uidx.at[pl.ds(base, _BPW)], ui)

  def astep(i, _):
    niv = ni[pl.ds(i * 16, 16)]
    for j in range(_NPN):
      npos[pl.ds(j * _BPW + i * 16, 16)] = niv * _NPN + j
    return 0
  lax.fori_loop(0, _BPW // 16, astep, 0, unroll=False)

  pltpu.async_copy(ndict.at[npos], nodes, sem).wait()

  def estep(i, _):
    pltpu.async_copy(ntab.at[ni.at[pl.ds(i * 8, 8)]], newsb, sem_e).wait()
    pltpu.sync_copy(newsb, news_g.at[pl.ds(base + i * 8, 8)])
    pltpu.async_copy(utab.at[ui.at[pl.ds(i * 8, 8)]], userb, sem_e).wait()
    pltpu.sync_copy(userb, user_g.at[pl.ds(base + i * 8, 8)])
    return 0
  lax.fori_loop(0, _BPW // 8, estep, 0, unroll=False)

  NCH = _SLOTS // _CS

  def build_and_fetch_adj(c):
    k3 = lax.rem(c, 3)
    nv = nodes[pl.ds(c * _CS, _CS)]
    for j in range(_ADJ):
      apos[pl.ds(k3 * (_CS * _ADJ) + j * _CS, _CS)] = nv * _ADJ + j
    pltpu.async_copy(adjf.at[apos.at[pl.ds(k3 * (_CS * _ADJ), _CS * _ADJ)]],
                     nvals.at[pl.ds(k3 * (_CS * _ADJ), _CS * _ADJ)],
                     sems_adj.at[k3])

  def fetch_emb(c):
    k3 = lax.rem(c, 3)
    k2 = lax.rem(c, 2)
    pltpu.make_async_copy(
        adjf.at[apos.at[pl.ds(k3 * (_CS * _ADJ), _CS * _ADJ)]],
        nvals.at[pl.ds(k3 * (_CS * _ADJ), _CS * _ADJ)],
        sems_adj.at[k3]).wait()
    pltpu.async_copy(embp.at[nvals.at[pl.ds(k3 * (_CS * _ADJ), _CS * _ADJ)]],
                     gbuf.at[k2], sems_g.at[k2])
    pltpu.async_copy(embp.at[nodes.at[pl.ds(c * _CS, _CS)]],
                     nbuf.at[k2], sems_n.at[k2])

  def out_copy(c):
    k2 = lax.rem(c, 2)
    jrow = c // (_BPW // _CS)
    b0 = lax.rem(c, _BPW // _CS) * _CS
    return pltpu.make_async_copy(
        obuf.at[k2], preact.at[jrow, pl.ds(base + b0, _CS)], sems_o.at[k2])

  build_and_fetch_adj(0)
  build_and_fetch_adj(1)
  fetch_emb(0)

  def dstep(c, _):
    k2 = lax.rem(c, 2)
    k3 = lax.rem(c, 3)

    @pl.when(c + 2 < NCH)
    def _():
      build_and_fetch_adj(c + 2)

    @pl.when(c + 1 < NCH)
    def _():
      fetch_emb(c + 1)

    pltpu.make_async_copy(
        embp.at[nvals.at[pl.ds(k3 * (_CS * _ADJ), _CS * _ADJ)]],
        gbuf.at[k2], sems_g.at[k2]).wait()
    pltpu.make_async_copy(embp.at[nodes.at[pl.ds(c * _CS, _CS)]],
                          nbuf.at[k2], sems_n.at[k2]).wait()

    @pl.when(c >= 2)
    def _():
      out_copy(c - 2).wait()

    def sstep(s, _):
      for v in range(_D // 16 + 1):
        sl = pl.ds(v * 16, 16)
        acc = gbuf[k2, s, sl]
        for j in range(1, _ADJ):
          acc = acc + gbuf[k2, j * _CS + s, sl]
        obuf[k2, s, sl] = nbuf[k2, s, sl] + acc * (1.0 / _ADJ)
      return 0
    lax.fori_loop(0, _CS, sstep, 0, unroll=False)
    out_copy(c).start()
    return 0
  lax.fori_loop(0, NCH, dstep, 0, unroll=False)
  out_copy(NCH - 2).wait()
  out_copy(NCH - 1).wait()


def _sc_gather(news_index, user_index, ndict_flat, adj_flat, embp,
               news_table, user_table):
  mesh = plsc.VectorSubcoreMesh(core_axis_name="c", subcore_axis_name="s",
                                num_cores=2, num_subcores=16)
  f = pl.kernel(
      _sc_gather_body,
      out_type=(
          jax.ShapeDtypeStruct((_B, _TITLEP), jnp.float32),
          jax.ShapeDtypeStruct((_B, _DP), jnp.float32),
          jax.ShapeDtypeStruct((_NPN, _B, _DP), jnp.float32),
      ),
      mesh=mesh,
      scratch_types=(
          pltpu.VMEM((_BPW,), jnp.int32),
          pltpu.VMEM((_BPW,), jnp.int32),
          pltpu.VMEM((_SLOTS,), jnp.int32),
          pltpu.VMEM((_SLOTS,), jnp.int32),
          pltpu.VMEM((3 * _CS * _ADJ,), jnp.int32),
          pltpu.VMEM((3 * _CS * _ADJ,), jnp.int32),
          pltpu.VMEM((2, _CS * _ADJ, _DP), jnp.float32),
          pltpu.VMEM((2, _CS, _DP), jnp.float32),
          pltpu.VMEM((2, _CS, _DP), jnp.float32),
          pltpu.VMEM((8, _TITLEP), jnp.float32),
          pltpu.VMEM((8, _DP), jnp.float32),
          pltpu.SemaphoreType.DMA,
          pltpu.SemaphoreType.DMA,
          pltpu.SemaphoreType.DMA((3,)),
          pltpu.SemaphoreType.DMA((2,)),
          pltpu.SemaphoreType.DMA((2,)),
          pltpu.SemaphoreType.DMA((2,)),
      ),
  )
  return f(news_index, user_index, ndict_flat, adj_flat, embp,
           news_table, user_table)


def _tc_transpad(xT, out_cols, rows_per_block):
  in_cols, rows = xT.shape
  nblk = (rows + rows_per_block - 1) // rows_per_block

  def body(x_ref, o_ref):
    o_ref[...] = jnp.concatenate(
        [x_ref[...].T,
         jnp.zeros((rows_per_block, out_cols - in_cols), jnp.float32)],
        axis=1)

  return pl.pallas_call(
      body,
      grid=(nblk,),
      in_specs=[pl.BlockSpec((in_cols, rows_per_block), lambda b: (0, b))],
      out_specs=pl.BlockSpec((rows_per_block, out_cols), lambda b: (b, 0)),
      out_shape=jax.ShapeDtypeStruct((rows, out_cols), jnp.float32),
  )(xT)


def _tc_transflat(xT, rows_per_block):
  c, rows = xT.shape
  nblk = (rows + rows_per_block - 1) // rows_per_block

  def body(x_ref, o_ref):
    o_ref[...] = x_ref[...].T.reshape(rows_per_block * c)

  return pl.pallas_call(
      body,
      grid=(nblk,),
      in_specs=[pl.BlockSpec((c, rows_per_block), lambda b: (0, b))],
      out_specs=pl.BlockSpec((rows_per_block * c,), lambda b: (b,)),
      out_shape=jax.ShapeDtypeStruct((rows * c,), jnp.int32),
  )(xT)


def _tc_pad(x, out_cols, rows_per_block):
  rows, in_cols = x.shape
  nblk = (rows + rows_per_block - 1) // rows_per_block

  def body(x_ref, o_ref):
    o_ref[...] = jnp.concatenate(
        [x_ref[...],
         jnp.zeros((rows_per_block, out_cols - in_cols), jnp.float32)],
        axis=1)

  return pl.pallas_call(
      body,
      grid=(nblk,),
      in_specs=[pl.BlockSpec((rows_per_block, in_cols), lambda b: (b, 0))],
      out_specs=pl.BlockSpec((rows_per_block, out_cols), lambda b: (b, 0)),
      out_shape=jax.ShapeDtypeStruct((rows, out_cols), jnp.float32),
  )(x)


_BM = 256


def _tc_mlp_body(news_ref, user_ref, pre_ref,
                 W1, b1, W2, b2, aW1t, aW1b, ab1, aW2, ab2, a3, ab3,
                 cW1t, cW1b, cb1, cW2, cb2, c3, cb3,
                 out_a, out_c):
  t = jnp.dot(news_ref[:, :_TITLE], W1[...],
              preferred_element_type=jnp.float32) + b1[...]
  t = jnp.where(t > 0, t, jnp.exp(t) - 1.0)
  ne = jnp.tanh(jnp.dot(t, W2[...],
                        preferred_element_type=jnp.float32) + b2[...])
  state = jnp.concatenate([ne, user_ref[:, :_D]], axis=1)
  sA = jnp.dot(state, aW1t[...],
               preferred_element_type=jnp.float32) + ab1[...]
  sC = jnp.dot(state, cW1t[...],
               preferred_element_type=jnp.float32) + cb1[...]
  sA12 = jnp.broadcast_to(sA[None, :, :], (_NPN, _BM, _D)).reshape(
      _NPN * _BM, _D)
  sC12 = jnp.broadcast_to(sC[None, :, :], (_NPN, _BM, _D)).reshape(
      _NPN * _BM, _D)

  act = jnp.tanh(pre_ref[...].reshape(_NPN * _BM, _DP)[:, :_D])

  ha = jnp.tanh(jnp.dot(act, aW1b[...],
                        preferred_element_type=jnp.float32) + sA12)
  oa = jnp.tanh(jnp.dot(ha, aW2[...],
                        preferred_element_type=jnp.float32) + ab2[...])
  pa = jax.nn.sigmoid(jnp.sum(oa * a3[...], axis=1) + ab3[0, 0])
  out_a[...] = pa.reshape(_NPN, _BM)

  hc = jnp.tanh(jnp.dot(act, cW1b[...],
                        preferred_element_type=jnp.float32) + sC12)
  oc = jnp.tanh(jnp.dot(hc, cW2[...],
                        preferred_element_type=jnp.float32) + cb2[...])
  pc = jax.nn.sigmoid(jnp.sum(oc * c3[...], axis=1) + cb3[0, 0])
  out_c[...] = pc.reshape(_NPN, _BM)


def _tc_mlp(news_g, user_g, pre3, W1, b1, W2, b2,
            aW1t, aW1b, ab1, aW2, ab2, a3, ab3,
            cW1t, cW1b, cb1, cW2, cb2, c3, cb3):
  nb = _B // _BM
  full = lambda shape: pl.BlockSpec(shape, lambda b: (0,) * len(shape))
  return pl.pallas_call(
      _tc_mlp_body,
      grid=(nb,),
      in_specs=[
          pl.BlockSpec((_BM, _TITLEP), lambda b: (b, 0)),
          pl.BlockSpec((_BM, _DP), lambda b: (b, 0)),
          pl.BlockSpec((_NPN, _BM, _DP), lambda b: (0, b, 0)),
          full((_TITLE, _D)), full((1, _D)), full((_D, _D)), full((1, _D)),
          full((2 * _D, _D)), full((_D, _D)), full((1, _D)),
          full((_D, _D)), full((1, _D)), full((1, _D)), full((1, 1)),
          full((2 * _D, _D)), full((_D, _D)), full((1, _D)),
          full((_D, _D)), full((1, _D)), full((1, _D)), full((1, 1)),
      ],
      out_specs=[
          pl.BlockSpec((_NPN, _BM), lambda b: (0, b)),
          pl.BlockSpec((_NPN, _BM), lambda b: (0, b)),
      ],
      out_shape=(
          jax.ShapeDtypeStruct((_NPN, _B), jnp.float32),
          jax.ShapeDtypeStruct((_NPN, _B), jnp.float32),
      ),
  )(news_g, user_g, pre3, W1, b1, W2, b2,
    aW1t, aW1b, ab1, aW2, ab2, a3, ab3,
    cW1t, cW1b, cb1, cW2, cb2, c3, cb3)


def kernel(news_index, user_index, news_node_dict, re_entity_adj, news_table,
           user_table, node_embedding, W1, b1, W2, b2, aW1, ab1, aW2, ab2,
           aW3, ab3, cW1, cb1, cW2, cb2, cW3, cb3):
  ni = news_index.astype(jnp.int32)
  ui = user_index.astype(jnp.int32)
  ndict_flat = jnp.maximum(news_node_dict.astype(jnp.int32).reshape(-1), 0)
  adj_flat = jnp.maximum(re_entity_adj.astype(jnp.int32).reshape(-1), 0)
  embp = _tc_transpad(node_embedding.T, _DP, 2048)
  ntabp = _tc_transpad(news_table.T, _TITLEP, 2048)
  utabp = _tc_transpad(user_table.T, _DP, 2048)

  news_g, user_g, pre3 = _sc_gather(ni, ui, ndict_flat, adj_flat, embp,
                                    ntabp, utabp)

  out_a, out_c = _tc_mlp(
      news_g, user_g, pre3,
      W1, b1.reshape(1, _D), W2, b2.reshape(1, _D),
      aW1[: 2 * _D], aW1[2 * _D:], ab1.reshape(1, _D),
      aW2, ab2.reshape(1, _D), aW3.reshape(1, _D), ab3.reshape(1, 1),
      cW1[: 2 * _D], cW1[2 * _D:], cb1.reshape(1, _D),
      cW2, cb2.reshape(1, _D), cW3.reshape(1, _D), cb3.reshape(1, 1),
  )
  act_probs = out_a.T.reshape(_B, _NPN, 1)
  q_actions = out_c.T.reshape(_B, _NPN, 1)
  return (act_probs, q_actions)


# speedup vs baseline: 7.7072x; 1.0239x over previous
import functools

import jax
import jax.numpy as jnp
from jax import lax
from jax.experimental import pallas as pl
from jax.experimental.pallas import tpu as pltpu
from jax.experimental.pallas import tpu_sc as plsc

_B = 4096
_NPN = 12
_ADJ = 20
_D = 100
_DP = 128
_TITLE = 400
_TITLEP = 512
_NW = 32
_BPW = _B // _NW
_SLOTS = _BPW * _NPN
_CS = 16


def _sc_gather_body(nidx, uidx, ndict, adjf, embp, ntab, utab,
                    news_g, user_g, preact,
                    ni, ui, dbuf, nodes, abuf, nvals,
                    gbuf, nbuf, obuf, newsb, userb,
                    sem, sem_e, sems_adj, sems_g, sems_n, sems_o):
  wid = lax.axis_index("s") * 2 + lax.axis_index("c")
  base = wid * _BPW
  iot = lax.iota(jnp.int32, 16)

  pltpu.sync_copy(nidx.at[pl.ds(base, _BPW)], ni)
  pltpu.sync_copy(uidx.at[pl.ds(base, _BPW)], ui)

  def astep(i, _):
    pltpu.async_copy(ndict.at[ni.at[pl.ds(i * 32, 32)]], dbuf, sem).wait()
    for j in range(_NPN):
      for h in range(2):
        nodes[pl.ds(j * _BPW + i * 32 + h * 16, 16)] = plsc.load_gather(
            dbuf, [iot + h * 16, iot * 0 + j])
    return 0
  lax.fori_loop(0, _BPW // 32, astep, 0, unroll=False)

  def estep(i, _):
    pltpu.async_copy(ntab.at[ni.at[pl.ds(i * 8, 8)]], newsb, sem_e).wait()
    pltpu.sync_copy(newsb, news_g.at[pl.ds(base + i * 8, 8)])
    pltpu.async_copy(utab.at[ui.at[pl.ds(i * 8, 8)]], userb, sem_e).wait()
    pltpu.sync_copy(userb, user_g.at[pl.ds(base + i * 8, 8)])
    return 0
  lax.fori_loop(0, _BPW // 8, estep, 0, unroll=False)

  NCH = _SLOTS // _CS

  def build_and_fetch_adj(c):
    k3 = lax.rem(c, 3)
    pltpu.async_copy(adjf.at[nodes.at[pl.ds(c * _CS, _CS)]],
                     abuf.at[pl.ds(k3 * _CS, _CS)], sems_adj.at[k3])

  def fetch_emb(c):
    k3 = lax.rem(c, 3)
    k2 = lax.rem(c, 2)
    pltpu.make_async_copy(adjf.at[nodes.at[pl.ds(c * _CS, _CS)]],
                          abuf.at[pl.ds(k3 * _CS, _CS)], sems_adj.at[k3]).wait()
    for j in range(_ADJ):
      nvals[pl.ds(k3 * (_CS * _ADJ) + j * _CS, _CS)] = plsc.load_gather(
          abuf, [k3 * _CS + iot, iot * 0 + j])
    pltpu.async_copy(embp.at[nvals.at[pl.ds(k3 * (_CS * _ADJ), _CS * _ADJ)]],
                     gbuf.at[k2], sems_g.at[k2])
    pltpu.async_copy(embp.at[nodes.at[pl.ds(c * _CS, _CS)]],
                     nbuf.at[k2], sems_n.at[k2])

  def out_copy(c):
    k2 = lax.rem(c, 2)
    jrow = c // (_BPW // _CS)
    b0 = lax.rem(c, _BPW // _CS) * _CS
    return pltpu.make_async_copy(
        obuf.at[k2], preact.at[jrow, pl.ds(base + b0, _CS)], sems_o.at[k2])

  build_and_fetch_adj(0)
  build_and_fetch_adj(1)
  fetch_emb(0)

  def dstep(c, _):
    k2 = lax.rem(c, 2)
    k3 = lax.rem(c, 3)

    @pl.when(c + 2 < NCH)
    def _():
      build_and_fetch_adj(c + 2)

    @pl.when(c + 1 < NCH)
    def _():
      fetch_emb(c + 1)

    pltpu.make_async_copy(
        embp.at[nvals.at[pl.ds(k3 * (_CS * _ADJ), _CS * _ADJ)]],
        gbuf.at[k2], sems_g.at[k2]).wait()
    pltpu.make_async_copy(embp.at[nodes.at[pl.ds(c * _CS, _CS)]],
                          nbuf.at[k2], sems_n.at[k2]).wait()

    @pl.when(c >= 2)
    def _():
      out_copy(c - 2).wait()

    def sstep(s, _):
      for v in range(_D // 16 + 1):
        sl = pl.ds(v * 16, 16)
        acc = gbuf[k2, s, sl]
        for j in range(1, _ADJ):
          acc = acc + gbuf[k2, j * _CS + s, sl]
        obuf[k2, s, sl] = nbuf[k2, s, sl] + acc * (1.0 / _ADJ)
      return 0
    lax.fori_loop(0, _CS, sstep, 0, unroll=False)
    out_copy(c).start()
    return 0
  lax.fori_loop(0, NCH, dstep, 0, unroll=False)
  out_copy(NCH - 2).wait()
  out_copy(NCH - 1).wait()


def _sc_gather(news_index, user_index, ndict_flat, adj_flat, embp,
               news_table, user_table):
  mesh = plsc.VectorSubcoreMesh(core_axis_name="c", subcore_axis_name="s",
                                num_cores=2, num_subcores=16)
  f = pl.kernel(
      _sc_gather_body,
      compiler_params=pltpu.CompilerParams(needs_layout_passes=False),
      out_type=(
          jax.ShapeDtypeStruct((_B, _TITLEP), jnp.float32),
          jax.ShapeDtypeStruct((_B, _DP), jnp.float32),
          jax.ShapeDtypeStruct((_NPN, _B, _DP), jnp.float32),
      ),
      mesh=mesh,
      scratch_types=(
          pltpu.VMEM((_BPW,), jnp.int32),
          pltpu.VMEM((_BPW,), jnp.int32),
          pltpu.VMEM((32, _DP), jnp.int32),
          pltpu.VMEM((_SLOTS,), jnp.int32),
          pltpu.VMEM((3 * _CS, _DP), jnp.int32),
          pltpu.VMEM((3 * _CS * _ADJ,), jnp.int32),
          pltpu.VMEM((2, _CS * _ADJ, _DP), jnp.float32),
          pltpu.VMEM((2, _CS, _DP), jnp.float32),
          pltpu.VMEM((2, _CS, _DP), jnp.float32),
          pltpu.VMEM((8, _TITLEP), jnp.float32),
          pltpu.VMEM((8, _DP), jnp.float32),
          pltpu.SemaphoreType.DMA,
          pltpu.SemaphoreType.DMA,
          pltpu.SemaphoreType.DMA((3,)),
          pltpu.SemaphoreType.DMA((2,)),
          pltpu.SemaphoreType.DMA((2,)),
          pltpu.SemaphoreType.DMA((2,)),
      ),
  )
  return f(news_index, user_index, ndict_flat, adj_flat, embp,
           news_table, user_table)


def _tc_transpad(xT, out_cols, rows_per_block):
  in_cols, rows = xT.shape
  dtype = xT.dtype
  nblk = (rows + rows_per_block - 1) // rows_per_block

  def body(x_ref, o_ref):
    o_ref[...] = jnp.concatenate(
        [x_ref[...].T,
         jnp.zeros((rows_per_block, out_cols - in_cols), dtype)],
        axis=1)

  return pl.pallas_call(
      body,
      grid=(nblk,),
      in_specs=[pl.BlockSpec((in_cols, rows_per_block), lambda b: (0, b))],
      out_specs=pl.BlockSpec((rows_per_block, out_cols), lambda b: (b, 0)),
      out_shape=jax.ShapeDtypeStruct((rows, out_cols), dtype),
  )(xT)


def _tc_pad(x, out_cols, rows_per_block):
  rows, in_cols = x.shape
  nblk = (rows + rows_per_block - 1) // rows_per_block

  def body(x_ref, o_ref):
    o_ref[...] = jnp.concatenate(
        [x_ref[...],
         jnp.zeros((rows_per_block, out_cols - in_cols), jnp.float32)],
        axis=1)

  return pl.pallas_call(
      body,
      grid=(nblk,),
      in_specs=[pl.BlockSpec((rows_per_block, in_cols), lambda b: (b, 0))],
      out_specs=pl.BlockSpec((rows_per_block, out_cols), lambda b: (b, 0)),
      out_shape=jax.ShapeDtypeStruct((rows, out_cols), jnp.float32),
  )(x)


_BM = 256


def _tc_mlp_body(news_ref, user_ref, pre_ref,
                 W1, b1, W2, b2, aW1t, aW1b, ab1, aW2, ab2, a3, ab3,
                 cW1t, cW1b, cb1, cW2, cb2, c3, cb3,
                 out_a, out_c):
  t = jnp.dot(news_ref[:, :_TITLE], W1[...],
              preferred_element_type=jnp.float32) + b1[...]
  t = jnp.where(t > 0, t, jnp.exp(t) - 1.0)
  ne = jnp.tanh(jnp.dot(t, W2[...],
                        preferred_element_type=jnp.float32) + b2[...])
  state = jnp.concatenate([ne, user_ref[:, :_D]], axis=1)
  sA = jnp.dot(state, aW1t[...],
               preferred_element_type=jnp.float32) + ab1[...]
  sC = jnp.dot(state, cW1t[...],
               preferred_element_type=jnp.float32) + cb1[...]
  sA12 = jnp.broadcast_to(sA[None, :, :], (_NPN, _BM, _D)).reshape(
      _NPN * _BM, _D)
  sC12 = jnp.broadcast_to(sC[None, :, :], (_NPN, _BM, _D)).reshape(
      _NPN * _BM, _D)

  act = jnp.tanh(pre_ref[...].reshape(_NPN * _BM, _DP)[:, :_D])

  ha = jnp.tanh(jnp.dot(act, aW1b[...],
                        preferred_element_type=jnp.float32) + sA12)
  oa = jnp.tanh(jnp.dot(ha, aW2[...],
                        preferred_element_type=jnp.float32) + ab2[...])
  pa = jax.nn.sigmoid(jnp.sum(oa * a3[...], axis=1) + ab3[0, 0])
  out_a[...] = pa.reshape(_NPN, _BM)

  hc = jnp.tanh(jnp.dot(act, cW1b[...],
                        preferred_element_type=jnp.float32) + sC12)
  oc = jnp.tanh(jnp.dot(hc, cW2[...],
                        preferred_element_type=jnp.float32) + cb2[...])
  pc = jax.nn.sigmoid(jnp.sum(oc * c3[...], axis=1) + cb3[0, 0])
  out_c[...] = pc.reshape(_NPN, _BM)


def _tc_mlp(news_g, user_g, pre3, W1, b1, W2, b2,
            aW1t, aW1b, ab1, aW2, ab2, a3, ab3,
            cW1t, cW1b, cb1, cW2, cb2, c3, cb3):
  nb = _B // _BM
  full = lambda shape: pl.BlockSpec(shape, lambda b: (0,) * len(shape))
  return pl.pallas_call(
      _tc_mlp_body,
      grid=(nb,),
      in_specs=[
          pl.BlockSpec((_BM, _TITLEP), lambda b: (b, 0)),
          pl.BlockSpec((_BM, _DP), lambda b: (b, 0)),
          pl.BlockSpec((_NPN, _BM, _DP), lambda b: (0, b, 0)),
          full((_TITLE, _D)), full((1, _D)), full((_D, _D)), full((1, _D)),
          full((2 * _D, _D)), full((_D, _D)), full((1, _D)),
          full((_D, _D)), full((1, _D)), full((1, _D)), full((1, 1)),
          full((2 * _D, _D)), full((_D, _D)), full((1, _D)),
          full((_D, _D)), full((1, _D)), full((1, _D)), full((1, 1)),
      ],
      out_specs=[
          pl.BlockSpec((_NPN, _BM), lambda b: (0, b)),
          pl.BlockSpec((_NPN, _BM), lambda b: (0, b)),
      ],
      out_shape=(
          jax.ShapeDtypeStruct((_NPN, _B), jnp.float32),
          jax.ShapeDtypeStruct((_NPN, _B), jnp.float32),
      ),
  )(news_g, user_g, pre3, W1, b1, W2, b2,
    aW1t, aW1b, ab1, aW2, ab2, a3, ab3,
    cW1t, cW1b, cb1, cW2, cb2, c3, cb3)


def kernel(news_index, user_index, news_node_dict, re_entity_adj, news_table,
           user_table, node_embedding, W1, b1, W2, b2, aW1, ab1, aW2, ab2,
           aW3, ab3, cW1, cb1, cW2, cb2, cW3, cb3):
  ni = news_index.astype(jnp.int32)
  ui = user_index.astype(jnp.int32)
  ndict128 = _tc_transpad(news_node_dict.astype(jnp.int32).T, _DP, 2048)
  adj128 = _tc_transpad(re_entity_adj.astype(jnp.int32).T, _DP, 2048)
  embp = _tc_transpad(node_embedding.T, _DP, 2048)
  ntabp = _tc_transpad(news_table.T, _TITLEP, 2048)
  utabp = _tc_transpad(user_table.T, _DP, 2048)

  news_g, user_g, pre3 = _sc_gather(ni, ui, ndict128, adj128, embp,
                                    ntabp, utabp)

  out_a, out_c = _tc_mlp(
      news_g, user_g, pre3,
      W1, b1.reshape(1, _D), W2, b2.reshape(1, _D),
      aW1[: 2 * _D], aW1[2 * _D:], ab1.reshape(1, _D),
      aW2, ab2.reshape(1, _D), aW3.reshape(1, _D), ab3.reshape(1, 1),
      cW1[: 2 * _D], cW1[2 * _D:], cb1.reshape(1, _D),
      cW2, cb2.reshape(1, _D), cW3.reshape(1, _D), cb3.reshape(1, 1),
  )
  act_probs = out_a.T.reshape(_B, _NPN, 1)
  q_actions = out_c.T.reshape(_B, _NPN, 1)
  return (act_probs, q_actions)


# speedup vs baseline: 7.7241x; 1.0022x over previous
import functools

import jax
import jax.numpy as jnp
from jax import lax
from jax.experimental import pallas as pl
from jax.experimental.pallas import tpu as pltpu
from jax.experimental.pallas import tpu_sc as plsc

_B = 4096
_NPN = 12
_ADJ = 20
_D = 100
_DP = 128
_TITLE = 400
_TITLEP = 512
_NW = 32
_BPW = _B // _NW
_SLOTS = _BPW * _NPN
_CS = 16


def _sc_gather_body(nidx, uidx, ndict, adjf, embp, ntab, utab,
                    news_g, user_g, preact,
                    ni, ui, dbuf, nodes, abuf, nvals,
                    gbuf, nbuf, obuf, newsb, userb,
                    sem, sems_news, sems_user, sems_adj, sems_g, sems_n,
                    sems_o):
  wid = lax.axis_index("s") * 2 + lax.axis_index("c")
  base = wid * _BPW
  iot = lax.iota(jnp.int32, 16)

  pltpu.sync_copy(nidx.at[pl.ds(base, _BPW)], ni)
  pltpu.sync_copy(uidx.at[pl.ds(base, _BPW)], ui)

  def astep(i, _):
    pltpu.async_copy(ndict.at[ni.at[pl.ds(i * 16, 16)]], dbuf, sem).wait()
    for j in range(_NPN):
      nodes[pl.ds(j * _BPW + i * 16, 16)] = plsc.load_gather(
          dbuf, [iot, iot * 0 + j])
    return 0
  lax.fori_loop(0, _BPW // 16, astep, 0, unroll=False)

  NEC = _BPW // 8

  def eu_issue(i):
    k2 = lax.rem(i, 2)
    pltpu.async_copy(ntab.at[ni.at[pl.ds(i * 8, 8)]], newsb.at[k2],
                     sems_news.at[k2])
    pltpu.async_copy(utab.at[ui.at[pl.ds(i * 8, 8)]], userb.at[k2],
                     sems_user.at[k2])

  def eu_drain(i):
    k2 = lax.rem(i, 2)
    pltpu.make_async_copy(ntab.at[ni.at[pl.ds(i * 8, 8)]], newsb.at[k2],
                          sems_news.at[k2]).wait()
    pltpu.sync_copy(newsb.at[k2], news_g.at[pl.ds(base + i * 8, 8)])
    pltpu.make_async_copy(utab.at[ui.at[pl.ds(i * 8, 8)]], userb.at[k2],
                          sems_user.at[k2]).wait()
    pltpu.sync_copy(userb.at[k2], user_g.at[pl.ds(base + i * 8, 8)])

  NCH = _SLOTS // _CS

  def build_and_fetch_adj(c):
    k3 = lax.rem(c, 3)
    pltpu.async_copy(adjf.at[nodes.at[pl.ds(c * _CS, _CS)]],
                     abuf.at[pl.ds(k3 * _CS, _CS)], sems_adj.at[k3])

  def fetch_emb(c):
    k3 = lax.rem(c, 3)
    k2 = lax.rem(c, 2)
    pltpu.make_async_copy(adjf.at[nodes.at[pl.ds(c * _CS, _CS)]],
                          abuf.at[pl.ds(k3 * _CS, _CS)], sems_adj.at[k3]).wait()
    for j in range(_ADJ):
      nvals[pl.ds(k3 * (_CS * _ADJ) + j * _CS, _CS)] = plsc.load_gather(
          abuf, [k3 * _CS + iot, iot * 0 + j])
    pltpu.async_copy(embp.at[nvals.at[pl.ds(k3 * (_CS * _ADJ), _CS * _ADJ)]],
                     gbuf.at[k2], sems_g.at[k2])
    pltpu.async_copy(embp.at[nodes.at[pl.ds(c * _CS, _CS)]],
                     nbuf.at[k2], sems_n.at[k2])

  def out_copy(c):
    k2 = lax.rem(c, 2)
    jrow = c // (_BPW // _CS)
    b0 = lax.rem(c, _BPW // _CS) * _CS
    return pltpu.make_async_copy(
        obuf.at[k2], preact.at[jrow, pl.ds(base + b0, _CS)], sems_o.at[k2])

  build_and_fetch_adj(0)
  build_and_fetch_adj(1)
  fetch_emb(0)
  eu_issue(0)

  def dstep(c, _):
    k2 = lax.rem(c, 2)
    k3 = lax.rem(c, 3)

    @pl.when(c + 2 < NCH)
    def _():
      build_and_fetch_adj(c + 2)

    @pl.when(c + 1 < NCH)
    def _():
      fetch_emb(c + 1)

    @pl.when(c + 1 < NEC)
    def _():
      eu_issue(c + 1)

    @pl.when(c < NEC)
    def _():
      eu_drain(c)

    pltpu.make_async_copy(
        embp.at[nvals.at[pl.ds(k3 * (_CS * _ADJ), _CS * _ADJ)]],
        gbuf.at[k2], sems_g.at[k2]).wait()
    pltpu.make_async_copy(embp.at[nodes.at[pl.ds(c * _CS, _CS)]],
                          nbuf.at[k2], sems_n.at[k2]).wait()

    @pl.when(c >= 2)
    def _():
      out_copy(c - 2).wait()

    def sstep(s, _):
      for v in range(_D // 16 + 1):
        sl = pl.ds(v * 16, 16)
        acc = gbuf[k2, s, sl]
        for j in range(1, _ADJ):
          acc = acc + gbuf[k2, j * _CS + s, sl]
        obuf[k2, s, sl] = nbuf[k2, s, sl] + acc * (1.0 / _ADJ)
      return 0
    lax.fori_loop(0, _CS, sstep, 0, unroll=False)
    out_copy(c).start()
    return 0
  lax.fori_loop(0, NCH, dstep, 0, unroll=False)
  out_copy(NCH - 2).wait()
  out_copy(NCH - 1).wait()


def _sc_gather(news_index, user_index, ndict_flat, adj_flat, embp,
               news_table, user_table):
  mesh = plsc.VectorSubcoreMesh(core_axis_name="c", subcore_axis_name="s",
                                num_cores=2, num_subcores=16)
  f = pl.kernel(
      _sc_gather_body,
      compiler_params=pltpu.CompilerParams(needs_layout_passes=False),
      out_type=(
          jax.ShapeDtypeStruct((_B, _TITLEP), jnp.float32),
          jax.ShapeDtypeStruct((_B, _DP), jnp.float32),
          jax.ShapeDtypeStruct((_NPN, _B, _DP), jnp.float32),
      ),
      mesh=mesh,
      scratch_types=(
          pltpu.VMEM((_BPW,), jnp.int32),
          pltpu.VMEM((_BPW,), jnp.int32),
          pltpu.VMEM((16, _DP), jnp.int32),
          pltpu.VMEM((_SLOTS,), jnp.int32),
          pltpu.VMEM((3 * _CS, _DP), jnp.int32),
          pltpu.VMEM((3 * _CS * _ADJ,), jnp.int32),
          pltpu.VMEM((2, _CS * _ADJ, _DP), jnp.float32),
          pltpu.VMEM((2, _CS, _DP), jnp.float32),
          pltpu.VMEM((2, _CS, _DP), jnp.float32),
          pltpu.VMEM((2, 8, _TITLEP), jnp.float32),
          pltpu.VMEM((2, 8, _DP), jnp.float32),
          pltpu.SemaphoreType.DMA,
          pltpu.SemaphoreType.DMA((2,)),
          pltpu.SemaphoreType.DMA((2,)),
          pltpu.SemaphoreType.DMA((3,)),
          pltpu.SemaphoreType.DMA((2,)),
          pltpu.SemaphoreType.DMA((2,)),
          pltpu.SemaphoreType.DMA((2,)),
      ),
  )
  return f(news_index, user_index, ndict_flat, adj_flat, embp,
           news_table, user_table)


def _tc_transpad(xT, out_cols, rows_per_block):
  in_cols, rows = xT.shape
  dtype = xT.dtype
  nblk = (rows + rows_per_block - 1) // rows_per_block

  def body(x_ref, o_ref):
    o_ref[...] = jnp.concatenate(
        [x_ref[...].T,
         jnp.zeros((rows_per_block, out_cols - in_cols), dtype)],
        axis=1)

  return pl.pallas_call(
      body,
      grid=(nblk,),
      in_specs=[pl.BlockSpec((in_cols, rows_per_block), lambda b: (0, b))],
      out_specs=pl.BlockSpec((rows_per_block, out_cols), lambda b: (b, 0)),
      out_shape=jax.ShapeDtypeStruct((rows, out_cols), dtype),
  )(xT)


def _tc_pad(x, out_cols, rows_per_block):
  rows, in_cols = x.shape
  nblk = (rows + rows_per_block - 1) // rows_per_block

  def body(x_ref, o_ref):
    o_ref[...] = jnp.concatenate(
        [x_ref[...],
         jnp.zeros((rows_per_block, out_cols - in_cols), jnp.float32)],
        axis=1)

  return pl.pallas_call(
      body,
      grid=(nblk,),
      in_specs=[pl.BlockSpec((rows_per_block, in_cols), lambda b: (b, 0))],
      out_specs=pl.BlockSpec((rows_per_block, out_cols), lambda b: (b, 0)),
      out_shape=jax.ShapeDtypeStruct((rows, out_cols), jnp.float32),
  )(x)


_BM = 256


def _tc_mlp_body(news_ref, user_ref, pre_ref,
                 W1, b1, W2, b2, aW1t, aW1b, ab1, aW2, ab2, a3, ab3,
                 cW1t, cW1b, cb1, cW2, cb2, c3, cb3,
                 out_a, out_c):
  t = jnp.dot(news_ref[:, :_TITLE], W1[...],
              preferred_element_type=jnp.float32) + b1[...]
  t = jnp.where(t > 0, t, jnp.exp(t) - 1.0)
  ne = jnp.tanh(jnp.dot(t, W2[...],
                        preferred_element_type=jnp.float32) + b2[...])
  state = jnp.concatenate([ne, user_ref[:, :_D]], axis=1)
  sA = jnp.dot(state, aW1t[...],
               preferred_element_type=jnp.float32) + ab1[...]
  sC = jnp.dot(state, cW1t[...],
               preferred_element_type=jnp.float32) + cb1[...]
  sA12 = jnp.broadcast_to(sA[None, :, :], (_NPN, _BM, _D)).reshape(
      _NPN * _BM, _D)
  sC12 = jnp.broadcast_to(sC[None, :, :], (_NPN, _BM, _D)).reshape(
      _NPN * _BM, _D)

  act = jnp.tanh(pre_ref[...].reshape(_NPN * _BM, _DP)[:, :_D])

  ha = jnp.tanh(jnp.dot(act, aW1b[...],
                        preferred_element_type=jnp.float32) + sA12)
  oa = jnp.tanh(jnp.dot(ha, aW2[...],
                        preferred_element_type=jnp.float32) + ab2[...])
  pa = jax.nn.sigmoid(jnp.sum(oa * a3[...], axis=1) + ab3[0, 0])
  out_a[...] = pa.reshape(_NPN, _BM)

  hc = jnp.tanh(jnp.dot(act, cW1b[...],
                        preferred_element_type=jnp.float32) + sC12)
  oc = jnp.tanh(jnp.dot(hc, cW2[...],
                        preferred_element_type=jnp.float32) + cb2[...])
  pc = jax.nn.sigmoid(jnp.sum(oc * c3[...], axis=1) + cb3[0, 0])
  out_c[...] = pc.reshape(_NPN, _BM)


def _tc_mlp(news_g, user_g, pre3, W1, b1, W2, b2,
            aW1t, aW1b, ab1, aW2, ab2, a3, ab3,
            cW1t, cW1b, cb1, cW2, cb2, c3, cb3):
  nb = _B // _BM
  full = lambda shape: pl.BlockSpec(shape, lambda b: (0,) * len(shape))
  return pl.pallas_call(
      _tc_mlp_body,
      grid=(nb,),
      in_specs=[
          pl.BlockSpec((_BM, _TITLEP), lambda b: (b, 0)),
          pl.BlockSpec((_BM, _DP), lambda b: (b, 0)),
          pl.BlockSpec((_NPN, _BM, _DP), lambda b: (0, b, 0)),
          full((_TITLE, _D)), full((1, _D)), full((_D, _D)), full((1, _D)),
          full((2 * _D, _D)), full((_D, _D)), full((1, _D)),
          full((_D, _D)), full((1, _D)), full((1, _D)), full((1, 1)),
          full((2 * _D, _D)), full((_D, _D)), full((1, _D)),
          full((_D, _D)), full((1, _D)), full((1, _D)), full((1, 1)),
      ],
      out_specs=[
          pl.BlockSpec((_NPN, _BM), lambda b: (0, b)),
          pl.BlockSpec((_NPN, _BM), lambda b: (0, b)),
      ],
      out_shape=(
          jax.ShapeDtypeStruct((_NPN, _B), jnp.float32),
          jax.ShapeDtypeStruct((_NPN, _B), jnp.float32),
      ),
  )(news_g, user_g, pre3, W1, b1, W2, b2,
    aW1t, aW1b, ab1, aW2, ab2, a3, ab3,
    cW1t, cW1b, cb1, cW2, cb2, c3, cb3)


def kernel(news_index, user_index, news_node_dict, re_entity_adj, news_table,
           user_table, node_embedding, W1, b1, W2, b2, aW1, ab1, aW2, ab2,
           aW3, ab3, cW1, cb1, cW2, cb2, cW3, cb3):
  ni = news_index.astype(jnp.int32)
  ui = user_index.astype(jnp.int32)
  ndict128 = _tc_transpad(news_node_dict.astype(jnp.int32).T, _DP, 2048)
  adj128 = _tc_transpad(re_entity_adj.astype(jnp.int32).T, _DP, 2048)
  embp = _tc_transpad(node_embedding.T, _DP, 2048)
  ntabp = _tc_transpad(news_table.T, _TITLEP, 2048)
  utabp = _tc_transpad(user_table.T, _DP, 2048)

  news_g, user_g, pre3 = _sc_gather(ni, ui, ndict128, adj128, embp,
                                    ntabp, utabp)

  out_a, out_c = _tc_mlp(
      news_g, user_g, pre3,
      W1, b1.reshape(1, _D), W2, b2.reshape(1, _D),
      aW1[: 2 * _D], aW1[2 * _D:], ab1.reshape(1, _D),
      aW2, ab2.reshape(1, _D), aW3.reshape(1, _D), ab3.reshape(1, 1),
      cW1[: 2 * _D], cW1[2 * _D:], cb1.reshape(1, _D),
      cW2, cb2.reshape(1, _D), cW3.reshape(1, _D), cb3.reshape(1, 1),
  )
  act_probs = out_a.T.reshape(_B, _NPN, 1)
  q_actions = out_c.T.reshape(_B, _NPN, 1)
  return (act_probs, q_actions)
